# baseline (speedup 1.0000x reference)
import jax
import jax.numpy as jnp
from jax.experimental import pallas as pl

_N = 10000
_K = 32
_D = 128
_S = 16
_B = 200
_KC = 8


def _agg_block(self_ref, neigh_ref, ws_ref, wn_ref, a_ref, out_ref):
    f32 = jnp.float32
    sv = self_ref[...]
    ws = ws_ref[...]
    wn = wn_ref[...]
    a_row = a_ref[...]

    h_self = jax.lax.dot_general(
        sv, ws, (((1,), (0,)), ((), ())), preferred_element_type=f32
    )
    w_a = jax.lax.dot_general(
        a_row, wn, (((1,), (1,)), ((), ())), preferred_element_type=f32
    )

    w_a3 = w_a.reshape(1, 1, _D)
    chunks = []
    for k0 in range(0, _K, _KC):
        prod = neigh_ref[:, k0 : k0 + _KC, :] * w_a3
        chunks.append(jnp.sum(prod, axis=-1))
    logit_n = jnp.maximum(jnp.concatenate(chunks, axis=1), 0.0)
    logit_s = jnp.maximum(
        jax.lax.dot_general(
            h_self, a_row, (((1,), (1,)), ((), ())), preferred_element_type=f32
        ),
        0.0,
    )

    m = jnp.maximum(jnp.max(logit_n, axis=1, keepdims=True), logit_s)
    e_n = jnp.exp(logit_n - m)
    e_s = jnp.exp(logit_s - m)
    z = e_s + jnp.sum(e_n, axis=1, keepdims=True)
    s_n = e_n / z

    k_iota = jax.lax.broadcasted_iota(jnp.int32, (1, _K), 1)
    rank = jnp.zeros(logit_n.shape, f32)
    for j in range(_K):
        lj = logit_n[:, j : j + 1]
        gt = (lj > logit_n).astype(f32)
        eq = jnp.where((lj == logit_n) & (j < k_iota), 1.0, 0.0)
        rank = rank + gt + eq
    w = s_n * (rank < _S).astype(f32)

    agg = jnp.zeros((_B, _D), f32)
    for k0 in range(0, _K, _KC):
        wc = w[:, k0 : k0 + _KC, None]
        agg = agg + jnp.sum(neigh_ref[:, k0 : k0 + _KC, :] * wc, axis=1)
    out = h_self + jax.lax.dot_general(
        agg, wn, (((1,), (0,)), ((), ())), preferred_element_type=f32
    )
    out_ref[...] = jnp.maximum(out, 0.0)


def kernel(self_vecs, neigh_vecs, self_weights, neigh_weights, attention_weights):
    a_row = attention_weights.reshape(1, _D)
    return pl.pallas_call(
        _agg_block,
        grid=(_N // _B,),
        in_specs=[
            pl.BlockSpec((_B, _D), lambda i: (i, 0)),
            pl.BlockSpec((_B, _K, _D), lambda i: (i, 0, 0)),
            pl.BlockSpec((_D, _D), lambda i: (0, 0)),
            pl.BlockSpec((_D, _D), lambda i: (0, 0)),
            pl.BlockSpec((1, _D), lambda i: (0, 0)),
        ],
        out_specs=pl.BlockSpec((_B, _D), lambda i: (i, 0)),
        out_shape=jax.ShapeDtypeStruct((_N, _D), jnp.float32),
    )(self_vecs, neigh_vecs, self_weights, neigh_weights, a_row)


# speedup vs baseline: 1.0139x; 1.0139x over previous
import jax
import jax.numpy as jnp
from jax.experimental import pallas as pl

_N = 10000
_K = 32
_D = 128
_S = 16
_B = 200
_KC = 16


def _agg_block(self_ref, neigh_ref, ws_ref, wn_ref, a_ref, out_ref):
    f32 = jnp.float32
    sv = self_ref[...]
    ws = ws_ref[...]
    wn = wn_ref[...]
    a_row = a_ref[...]

    h_self = jax.lax.dot_general(
        sv, ws, (((1,), (0,)), ((), ())), preferred_element_type=f32
    )
    w_a = jax.lax.dot_general(
        a_row, wn, (((1,), (1,)), ((), ())), preferred_element_type=f32
    )

    w_a3 = w_a.reshape(1, 1, _D)
    chunks = []
    for k0 in range(0, _K, _KC):
        prod = neigh_ref[:, k0 : k0 + _KC, :] * w_a3
        chunks.append(jnp.sum(prod, axis=-1))
    logit_n = jnp.maximum(jnp.concatenate(chunks, axis=1), 0.0)
    logit_s = jnp.maximum(
        jax.lax.dot_general(
            h_self, a_row, (((1,), (1,)), ((), ())), preferred_element_type=f32
        ),
        0.0,
    )

    m = jnp.maximum(jnp.max(logit_n, axis=1, keepdims=True), logit_s)
    e_n = jnp.exp(logit_n - m)
    e_s = jnp.exp(logit_s - m)
    z = e_s + jnp.sum(e_n, axis=1, keepdims=True)
    s_n = e_n / z

    k_iota = jax.lax.broadcasted_iota(jnp.int32, (1, _K), 1)
    rank = jnp.zeros(logit_n.shape, f32)
    for j in range(_K):
        lj = logit_n[:, j : j + 1]
        gt = (lj > logit_n).astype(f32)
        eq = jnp.where((lj == logit_n) & (j < k_iota), 1.0, 0.0)
        rank = rank + gt + eq
    w = s_n * (rank < _S).astype(f32)

    agg = jnp.zeros((_B, _D), f32)
    for k0 in range(0, _K, _KC):
        wc = w[:, k0 : k0 + _KC, None]
        agg = agg + jnp.sum(neigh_ref[:, k0 : k0 + _KC, :] * wc, axis=1)
    out = h_self + jax.lax.dot_general(
        agg, wn, (((1,), (0,)), ((), ())), preferred_element_type=f32
    )
    out_ref[...] = jnp.maximum(out, 0.0)


def kernel(self_vecs, neigh_vecs, self_weights, neigh_weights, attention_weights):
    a_row = attention_weights.reshape(1, _D)
    return pl.pallas_call(
        _agg_block,
        grid=(_N // _B,),
        in_specs=[
            pl.BlockSpec((_B, _D), lambda i: (i, 0)),
            pl.BlockSpec((_B, _K, _D), lambda i: (i, 0, 0)),
            pl.BlockSpec((_D, _D), lambda i: (0, 0)),
            pl.BlockSpec((_D, _D), lambda i: (0, 0)),
            pl.BlockSpec((1, _D), lambda i: (0, 0)),
        ],
        out_specs=pl.BlockSpec((_B, _D), lambda i: (i, 0)),
        out_shape=jax.ShapeDtypeStruct((_N, _D), jnp.float32),
    )(self_vecs, neigh_vecs, self_weights, neigh_weights, a_row)


# speedup vs baseline: 1.1036x; 1.0884x over previous
import jax
import jax.numpy as jnp
from jax.experimental import pallas as pl

_N = 10000
_K = 32
_D = 128
_S = 16
_B = 400
_KC = 16


def _agg_block(self_ref, neigh_ref, ws_ref, wn_ref, a_ref, out_ref):
    f32 = jnp.float32
    sv = self_ref[...]
    ws = ws_ref[...]
    wn = wn_ref[...]
    a_row = a_ref[...]

    h_self = jax.lax.dot_general(
        sv, ws, (((1,), (0,)), ((), ())), preferred_element_type=f32
    )
    w_a = jax.lax.dot_general(
        a_row, wn, (((1,), (1,)), ((), ())), preferred_element_type=f32
    )

    w_a3 = w_a.reshape(1, 1, _D)
    chunks = []
    for k0 in range(0, _K, _KC):
        prod = neigh_ref[:, k0 : k0 + _KC, :] * w_a3
        chunks.append(jnp.sum(prod, axis=-1))
    logit_n = jnp.maximum(jnp.concatenate(chunks, axis=1), 0.0)
    logit_s = jnp.maximum(
        jax.lax.dot_general(
            h_self, a_row, (((1,), (1,)), ((), ())), preferred_element_type=f32
        ),
        0.0,
    )

    m = jnp.maximum(jnp.max(logit_n, axis=1, keepdims=True), logit_s)
    e_n = jnp.exp(logit_n - m)
    e_s = jnp.exp(logit_s - m)
    z = e_s + jnp.sum(e_n, axis=1, keepdims=True)
    s_n = e_n / z

    k_iota = jax.lax.broadcasted_iota(jnp.int32, (1, _K), 1)
    rank = jnp.zeros(logit_n.shape, f32)
    for j in range(_K):
        lj = logit_n[:, j : j + 1]
        gt = (lj > logit_n).astype(f32)
        eq = jnp.where((lj == logit_n) & (j < k_iota), 1.0, 0.0)
        rank = rank + gt + eq
    w = s_n * (rank < _S).astype(f32)

    agg = jnp.zeros((_B, _D), f32)
    for k0 in range(0, _K, _KC):
        wc = w[:, k0 : k0 + _KC, None]
        agg = agg + jnp.sum(neigh_ref[:, k0 : k0 + _KC, :] * wc, axis=1)
    out = h_self + jax.lax.dot_general(
        agg, wn, (((1,), (0,)), ((), ())), preferred_element_type=f32
    )
    out_ref[...] = jnp.maximum(out, 0.0)


def kernel(self_vecs, neigh_vecs, self_weights, neigh_weights, attention_weights):
    a_row = attention_weights.reshape(1, _D)
    return pl.pallas_call(
        _agg_block,
        grid=(_N // _B,),
        in_specs=[
            pl.BlockSpec((_B, _D), lambda i: (i, 0)),
            pl.BlockSpec((_B, _K, _D), lambda i: (i, 0, 0)),
            pl.BlockSpec((_D, _D), lambda i: (0, 0)),
            pl.BlockSpec((_D, _D), lambda i: (0, 0)),
            pl.BlockSpec((1, _D), lambda i: (0, 0)),
        ],
        out_specs=pl.BlockSpec((_B, _D), lambda i: (i, 0)),
        out_shape=jax.ShapeDtypeStruct((_N, _D), jnp.float32),
    )(self_vecs, neigh_vecs, self_weights, neigh_weights, a_row)
